# Initial kernel scaffold; baseline (speedup 1.0000x reference)
import jax
import jax.numpy as jnp
from jax.experimental import pallas as pl


def kernel(merged_feats, source_map_0, source_map_1):
    raise NotImplementedError("write your pallas kernel here")


# speedup vs baseline: 8489.8960x; 8489.8960x over previous
import jax
import jax.numpy as jnp
from jax import lax
from jax.experimental import pallas as pl
from jax.experimental.pallas import tpu as pltpu
from jax.experimental.pallas import tpu_sc as plsc

_B, _M1, _D = 32, 144, 768
_N1, _N0 = 288, 576
_L = 16
_R = 8
_NPAIR = _N0 // (2 * _R)


def _unmerge_body(x_hbm, sm0_hbm, sm1_hbm, out_hbm,
                  x_v, sm0_v, sm1_v, c0_v, c1_v, g_v, s_v, stage_v,
                  in_sem, out_sem0, out_sem1):
    b = lax.axis_index("s") * 2 + lax.axis_index("c")

    in_cp = pltpu.async_copy(x_hbm.at[b], x_v, in_sem)
    pltpu.sync_copy(sm0_hbm.at[b], sm0_v)
    pltpu.sync_copy(sm1_hbm.at[b], sm1_v)

    zeros = jnp.zeros((_L,), jnp.float32)
    for i in range(_M1 // _L):
        c1_v[pl.ds(i * _L, _L)] = zeros
    for i in range(_N1 // _L):
        c0_v[pl.ds(i * _L, _L)] = zeros

    for i in range(_N1 // _L):
        v = sm1_v[pl.ds(i * _L, _L)]
        cnt, last = plsc.scan_count(v)
        plsc.addupdate_scatter(c1_v, [v], cnt.astype(jnp.float32), mask=last)
    for i in range(_N0 // _L):
        v = sm0_v[pl.ds(i * _L, _L)]
        cnt, last = plsc.scan_count(v)
        plsc.addupdate_scatter(c0_v, [v], cnt.astype(jnp.float32), mask=last)

    @plsc.parallel_loop(0, _N0 // _L, unroll=4)
    def _(i):
        j = sm0_v[pl.ds(i * _L, _L)]
        jj = plsc.load_gather(sm1_v, [j])
        c0g = plsc.load_gather(c0_v, [j])
        c1g = plsc.load_gather(c1_v, [jj])
        g_v[pl.ds(i * _L, _L)] = jj * _D
        s_v[pl.ds(i * _L, _L)] = 1.0 / (c0g * c1g)

    in_cp.wait()

    cols = [lax.iota(jnp.int32, _L) + k * _L for k in range(_D // _L)]

    def emit_chunk(ci, buf):
        @plsc.parallel_loop(0, _R, unroll=_R)
        def _(r):
            n = ci * _R + r
            nv = lax.broadcast(n, (_L,))
            base = plsc.load_gather(g_v, [nv])
            sc = plsc.load_gather(s_v, [nv])
            for k in range(_D // _L):
                xv = plsc.load_gather(x_v, [base + cols[k]])
                stage_v[buf, r, pl.ds(k * _L, _L)] = xv * sc

    def drain(buf, sem):
        pltpu.make_async_copy(
            stage_v.at[buf], out_hbm.at[b, pl.ds(0, _R)], sem).wait()

    @pl.loop(0, _NPAIR)
    def _(i):
        @pl.when(i > 0)
        def _():
            drain(0, out_sem0)
        emit_chunk(2 * i, 0)
        pltpu.async_copy(
            stage_v.at[0], out_hbm.at[b, pl.ds((2 * i) * _R, _R)], out_sem0)

        @pl.when(i > 0)
        def _():
            drain(1, out_sem1)
        emit_chunk(2 * i + 1, 1)
        pltpu.async_copy(
            stage_v.at[1], out_hbm.at[b, pl.ds((2 * i + 1) * _R, _R)],
            out_sem1)

    drain(0, out_sem0)
    drain(1, out_sem1)


@jax.jit
def _unmerge(x, sm0, sm1):
    mesh = plsc.VectorSubcoreMesh(core_axis_name="c", subcore_axis_name="s")
    f = pl.kernel(
        _unmerge_body,
        out_type=jax.ShapeDtypeStruct((_B, _N0, _D), jnp.float32),
        mesh=mesh,
        compiler_params=pltpu.CompilerParams(needs_layout_passes=False),
        scratch_types=[
            pltpu.VMEM((_M1 * _D,), jnp.float32),
            pltpu.VMEM((_N0,), jnp.int32),
            pltpu.VMEM((_N1,), jnp.int32),
            pltpu.VMEM((_N1,), jnp.float32),
            pltpu.VMEM((_M1,), jnp.float32),
            pltpu.VMEM((_N0,), jnp.int32),
            pltpu.VMEM((_N0,), jnp.float32),
            pltpu.VMEM((2, _R, _D), jnp.float32),
            pltpu.SemaphoreType.DMA,
            pltpu.SemaphoreType.DMA,
            pltpu.SemaphoreType.DMA,
        ],
    )
    return f(x.reshape(_B, _M1 * _D), sm0, sm1)


def kernel(merged_feats, source_map_0, source_map_1):
    return _unmerge(merged_feats,
                    source_map_0.astype(jnp.int32),
                    source_map_1.astype(jnp.int32))


# speedup vs baseline: 19881.5108x; 2.3418x over previous
import jax
import jax.numpy as jnp
from jax import lax
from jax.experimental import pallas as pl
from jax.experimental.pallas import tpu as pltpu
from jax.experimental.pallas import tpu_sc as plsc

_B, _M1, _D = 32, 144, 768
_N1, _N0 = 288, 576
_L = 16
_R = 8
_NPAIR = _N0 // (2 * _R)


def _unmerge_body(x_hbm, sm0_hbm, sm1_hbm, out_hbm,
                  x_v, sm0_v, sm1_v, c0_v, c1_v, g_v, s_v, stage_v,
                  in_sem, out_sem0, out_sem1):
    b = lax.axis_index("s") * 2 + lax.axis_index("c")

    in_cp = pltpu.async_copy(x_hbm.at[b], x_v, in_sem)
    pltpu.sync_copy(sm0_hbm.at[b], sm0_v)
    pltpu.sync_copy(sm1_hbm.at[b], sm1_v)

    zeros = jnp.zeros((_L,), jnp.float32)
    for i in range(_M1 // _L):
        c1_v[pl.ds(i * _L, _L)] = zeros
    for i in range(_N1 // _L):
        c0_v[pl.ds(i * _L, _L)] = zeros

    for i in range(_N1 // _L):
        v = sm1_v[pl.ds(i * _L, _L)]
        cnt, last = plsc.scan_count(v)
        plsc.addupdate_scatter(c1_v, [v], cnt.astype(jnp.float32), mask=last)
    for i in range(_N0 // _L):
        v = sm0_v[pl.ds(i * _L, _L)]
        cnt, last = plsc.scan_count(v)
        plsc.addupdate_scatter(c0_v, [v], cnt.astype(jnp.float32), mask=last)

    @plsc.parallel_loop(0, _N0 // _L, unroll=4)
    def _(i):
        j = sm0_v[pl.ds(i * _L, _L)]
        jj = plsc.load_gather(sm1_v, [j])
        c0g = plsc.load_gather(c0_v, [j])
        c1g = plsc.load_gather(c1_v, [jj])
        g_v[pl.ds(i * _L, _L)] = jj
        s_v[pl.ds(i * _L, _L)] = 1.0 / (c0g * c1g)

    in_cp.wait()

    def emit_chunk(ci, buf):
        rows = []
        scs = []
        for r in range(_R):
            nv = lax.broadcast(ci * _R + r, (_L,))
            rows.append(jnp.max(plsc.load_gather(g_v, [nv])))
            scs.append(plsc.load_gather(s_v, [nv]))

        @plsc.parallel_loop(0, _D // _L, unroll=2)
        def _(k):
            col = k * _L
            for r in range(_R):
                xv = x_v[rows[r], pl.ds(col, _L)]
                stage_v[buf, r, pl.ds(col, _L)] = xv * scs[r]

    def drain(buf, sem):
        pltpu.make_async_copy(
            stage_v.at[buf], out_hbm.at[b, pl.ds(0, _R)], sem).wait()

    @pl.loop(0, _NPAIR)
    def _(i):
        @pl.when(i > 0)
        def _():
            drain(0, out_sem0)
        emit_chunk(2 * i, 0)
        pltpu.async_copy(
            stage_v.at[0], out_hbm.at[b, pl.ds((2 * i) * _R, _R)], out_sem0)

        @pl.when(i > 0)
        def _():
            drain(1, out_sem1)
        emit_chunk(2 * i + 1, 1)
        pltpu.async_copy(
            stage_v.at[1], out_hbm.at[b, pl.ds((2 * i + 1) * _R, _R)],
            out_sem1)

    drain(0, out_sem0)
    drain(1, out_sem1)


@jax.jit
def _unmerge(x, sm0, sm1):
    mesh = plsc.VectorSubcoreMesh(core_axis_name="c", subcore_axis_name="s")
    f = pl.kernel(
        _unmerge_body,
        out_type=jax.ShapeDtypeStruct((_B, _N0, _D), jnp.float32),
        mesh=mesh,
        compiler_params=pltpu.CompilerParams(needs_layout_passes=False),
        scratch_types=[
            pltpu.VMEM((_M1, _D), jnp.float32),
            pltpu.VMEM((_N0,), jnp.int32),
            pltpu.VMEM((_N1,), jnp.int32),
            pltpu.VMEM((_N1,), jnp.float32),
            pltpu.VMEM((_M1,), jnp.float32),
            pltpu.VMEM((_N0,), jnp.int32),
            pltpu.VMEM((_N0,), jnp.float32),
            pltpu.VMEM((2, _R, _D), jnp.float32),
            pltpu.SemaphoreType.DMA,
            pltpu.SemaphoreType.DMA,
            pltpu.SemaphoreType.DMA,
        ],
    )
    return f(x, sm0, sm1)


def kernel(merged_feats, source_map_0, source_map_1):
    return _unmerge(merged_feats,
                    source_map_0.astype(jnp.int32),
                    source_map_1.astype(jnp.int32))


# speedup vs baseline: 21064.5546x; 1.0595x over previous
import jax
import jax.numpy as jnp
from jax import lax
from jax.experimental import pallas as pl
from jax.experimental.pallas import tpu as pltpu
from jax.experimental.pallas import tpu_sc as plsc

_B, _M1, _D = 32, 144, 768
_N1, _N0 = 288, 576
_L = 16
_R = 8
_NPAIR = _N0 // (2 * _R)


def _unmerge_body(x_hbm, sm0_hbm, sm1_hbm, out_hbm,
                  x_v, sm0_v, sm1_v, c0_v, c1_v, g_v, s_v, stage_v,
                  in_sem, out_sem0, out_sem1):
    b = lax.axis_index("s") * 2 + lax.axis_index("c")

    in_cp = pltpu.async_copy(x_hbm.at[b], x_v, in_sem)
    pltpu.sync_copy(sm0_hbm.at[b], sm0_v)
    pltpu.sync_copy(sm1_hbm.at[b], sm1_v)

    zeros = jnp.zeros((_L,), jnp.float32)
    for i in range(_M1 // _L):
        c1_v[pl.ds(i * _L, _L)] = zeros
    for i in range(_N1 // _L):
        c0_v[pl.ds(i * _L, _L)] = zeros

    for i in range(_N1 // _L):
        v = sm1_v[pl.ds(i * _L, _L)]
        cnt, last = plsc.scan_count(v)
        plsc.addupdate_scatter(c1_v, [v], cnt.astype(jnp.float32), mask=last)
    for i in range(_N0 // _L):
        v = sm0_v[pl.ds(i * _L, _L)]
        cnt, last = plsc.scan_count(v)
        plsc.addupdate_scatter(c0_v, [v], cnt.astype(jnp.float32), mask=last)

    @plsc.parallel_loop(0, _N0 // _L, unroll=4)
    def _(i):
        j = sm0_v[pl.ds(i * _L, _L)]
        jj = plsc.load_gather(sm1_v, [j])
        c0g = plsc.load_gather(c0_v, [j])
        c1g = plsc.load_gather(c1_v, [jj])
        g_v[pl.ds(i * _L, _L)] = jj
        s_v[pl.ds(i * _L, _L)] = 1.0 / (c0g * c1g)

    in_cp.wait()

    def emit_chunk(ci, buf):
        rows = []
        scs = []
        for r in range(_R):
            nv = lax.broadcast(ci * _R + r, (_L,))
            rows.append(jnp.max(plsc.load_gather(g_v, [nv])))
            scs.append(plsc.load_gather(s_v, [nv]))

        @plsc.parallel_loop(0, _D // _L, unroll=4)
        def _(k):
            col = k * _L
            for r in range(_R):
                xv = x_v[rows[r], pl.ds(col, _L)]
                stage_v[buf, r, pl.ds(col, _L)] = xv * scs[r]

    def drain(buf, sem):
        pltpu.make_async_copy(
            stage_v.at[buf], out_hbm.at[b, pl.ds(0, _R)], sem).wait()

    @pl.loop(0, _NPAIR)
    def _(i):
        @pl.when(i > 0)
        def _():
            drain(0, out_sem0)
        emit_chunk(2 * i, 0)
        pltpu.async_copy(
            stage_v.at[0], out_hbm.at[b, pl.ds((2 * i) * _R, _R)], out_sem0)

        @pl.when(i > 0)
        def _():
            drain(1, out_sem1)
        emit_chunk(2 * i + 1, 1)
        pltpu.async_copy(
            stage_v.at[1], out_hbm.at[b, pl.ds((2 * i + 1) * _R, _R)],
            out_sem1)

    drain(0, out_sem0)
    drain(1, out_sem1)


@jax.jit
def _unmerge(x, sm0, sm1):
    mesh = plsc.VectorSubcoreMesh(core_axis_name="c", subcore_axis_name="s")
    f = pl.kernel(
        _unmerge_body,
        out_type=jax.ShapeDtypeStruct((_B, _N0, _D), jnp.float32),
        mesh=mesh,
        compiler_params=pltpu.CompilerParams(needs_layout_passes=False),
        scratch_types=[
            pltpu.VMEM((_M1, _D), jnp.float32),
            pltpu.VMEM((_N0,), jnp.int32),
            pltpu.VMEM((_N1,), jnp.int32),
            pltpu.VMEM((_N1,), jnp.float32),
            pltpu.VMEM((_M1,), jnp.float32),
            pltpu.VMEM((_N0,), jnp.int32),
            pltpu.VMEM((_N0,), jnp.float32),
            pltpu.VMEM((2, _R, _D), jnp.float32),
            pltpu.SemaphoreType.DMA,
            pltpu.SemaphoreType.DMA,
            pltpu.SemaphoreType.DMA,
        ],
    )
    return f(x, sm0, sm1)


def kernel(merged_feats, source_map_0, source_map_1):
    return _unmerge(merged_feats,
                    source_map_0.astype(jnp.int32),
                    source_map_1.astype(jnp.int32))
